# baseline (device time: 16797 ns/iter reference)
import jax
import jax.numpy as jnp
from jax import lax
from jax.experimental import pallas as pl
from jax.experimental.pallas import tpu as pltpu

SIZES = (224, 160, 96, 32)
C = len(SIZES)
OFFS = tuple(sum(SIZES[:i]) for i in range(C))


def kernel(x):
    m, n = x.shape
    half = m // 2
    assert sum(SIZES) == half

    def body(x_hbm, out_hbm, xl, sendy, ybuf, xrecv_buf, sums,
             lsem, osem, ysend, yrecv, xsend, xrecv, sync_x):
        my_x = lax.axis_index("x")
        my_y = lax.axis_index("y")
        my_z = lax.axis_index("z")
        yp = (my_x, 1 - my_y, my_z)
        xn = (1 - my_x, my_y, my_z)
        base = my_x * half

        barrier_sem = pltpu.get_barrier_semaphore()
        pl.semaphore_signal(barrier_sem, inc=1, device_id=yp,
                            device_id_type=pl.DeviceIdType.MESH)
        pl.semaphore_signal(sync_x, inc=1, device_id=xn,
                            device_id_type=pl.DeviceIdType.MESH)

        stage = pltpu.make_async_copy(
            x_hbm.at[pl.ds(base, half), :], xl, lsem)
        stage.start()

        stage.wait()
        for i in range(C):
            sl = pl.ds(OFFS[i], SIZES[i])
            sendy[sl, :] = xl[sl, :].astype(jnp.bfloat16)

        pl.semaphore_wait(barrier_sem, 1)

        rdma_y = []
        for i in range(C):
            sl = pl.ds(OFFS[i], SIZES[i])
            r = pltpu.make_async_remote_copy(
                src_ref=sendy.at[sl, :], dst_ref=ybuf.at[sl, :],
                send_sem=ysend.at[i], recv_sem=yrecv.at[i],
                device_id=yp, device_id_type=pl.DeviceIdType.MESH,
            )
            r.start()
            rdma_y.append(r)

        pl.semaphore_wait(sync_x, 1)

        rdma_x = []
        for i in range(C):
            sl = pl.ds(OFFS[i], SIZES[i])
            rdma_y[i].wait_recv()
            sums[sl, :] = sendy[sl, :] + ybuf[sl, :]
            r = pltpu.make_async_remote_copy(
                src_ref=sums.at[sl, :], dst_ref=xrecv_buf.at[sl, :],
                send_sem=xsend.at[i], recv_sem=xrecv.at[i],
                device_id=xn, device_id_type=pl.DeviceIdType.MESH,
            )
            r.start()
            rdma_x.append(r)

        sums_out = pltpu.make_async_copy(
            sums, out_hbm.at[pl.ds(base, half), :], osem.at[C])
        sums_out.start()

        other = (1 - my_x) * half
        odmas = []
        for i in range(C):
            sl = pl.ds(OFFS[i], SIZES[i])
            rdma_x[i].wait_recv()
            od = pltpu.make_async_copy(
                xrecv_buf.at[sl, :],
                out_hbm.at[pl.ds(other + OFFS[i], SIZES[i]), :],
                osem.at[i])
            od.start()
            odmas.append(od)

        sums_out.wait()
        for od in odmas:
            od.wait()
        for i in range(C):
            rdma_y[i].wait_send()
            rdma_x[i].wait_send()

    return pl.pallas_call(
        body,
        out_shape=jax.ShapeDtypeStruct((m, n), jnp.bfloat16),
        in_specs=[pl.BlockSpec(memory_space=pl.ANY)],
        out_specs=pl.BlockSpec(memory_space=pl.ANY),
        scratch_shapes=[
            pltpu.VMEM((half, n), jnp.float32),
            pltpu.VMEM((half, n), jnp.bfloat16),
            pltpu.VMEM((half, n), jnp.bfloat16),
            pltpu.VMEM((half, n), jnp.bfloat16),
            pltpu.VMEM((half, n), jnp.bfloat16),
            pltpu.SemaphoreType.DMA,
            pltpu.SemaphoreType.DMA((C + 1,)),
            pltpu.SemaphoreType.DMA((C,)),
            pltpu.SemaphoreType.DMA((C,)),
            pltpu.SemaphoreType.DMA((C,)),
            pltpu.SemaphoreType.DMA((C,)),
            pltpu.SemaphoreType.REGULAR,
        ],
        compiler_params=pltpu.CompilerParams(collective_id=0),
    )(x)


# device time: 15856 ns/iter; 1.0593x vs baseline; 1.0593x over previous
import jax
import jax.numpy as jnp
from jax import lax
from jax.experimental import pallas as pl
from jax.experimental.pallas import tpu as pltpu

SIZES = (128, 128, 128, 128)
C = len(SIZES)
OFFS = tuple(sum(SIZES[:i]) for i in range(C))


def kernel(x):
    m, n = x.shape
    half = m // 2
    assert sum(SIZES) == half

    def body(x_hbm, out_hbm, xl, sendy, ybuf, xrecv_buf, sums,
             lsem, osem, ysend, yrecv, xsend, xrecv, sync_x):
        my_x = lax.axis_index("x")
        my_y = lax.axis_index("y")
        my_z = lax.axis_index("z")
        yp = (my_x, 1 - my_y, my_z)
        xn = (1 - my_x, my_y, my_z)
        base = my_x * half

        barrier_sem = pltpu.get_barrier_semaphore()
        pl.semaphore_signal(barrier_sem, inc=1, device_id=yp,
                            device_id_type=pl.DeviceIdType.MESH)
        pl.semaphore_signal(sync_x, inc=1, device_id=xn,
                            device_id_type=pl.DeviceIdType.MESH)

        stage = pltpu.make_async_copy(
            x_hbm.at[pl.ds(base, half), :], xl, lsem)
        stage.start()

        stage.wait()
        for i in range(C):
            sl = pl.ds(OFFS[i], SIZES[i])
            sendy[sl, :] = xl[sl, :].astype(jnp.bfloat16)

        pl.semaphore_wait(barrier_sem, 1)

        rdma_y = []
        for i in range(C):
            sl = pl.ds(OFFS[i], SIZES[i])
            r = pltpu.make_async_remote_copy(
                src_ref=sendy.at[sl, :], dst_ref=ybuf.at[sl, :],
                send_sem=ysend.at[i], recv_sem=yrecv.at[i],
                device_id=yp, device_id_type=pl.DeviceIdType.MESH,
            )
            r.start()
            rdma_y.append(r)

        pl.semaphore_wait(sync_x, 1)

        rdma_x = []
        for i in range(C):
            sl = pl.ds(OFFS[i], SIZES[i])
            rdma_y[i].wait_recv()
            sums[sl, :] = sendy[sl, :] + ybuf[sl, :]
            r = pltpu.make_async_remote_copy(
                src_ref=sums.at[sl, :], dst_ref=xrecv_buf.at[sl, :],
                send_sem=xsend.at[i], recv_sem=xrecv.at[i],
                device_id=xn, device_id_type=pl.DeviceIdType.MESH,
            )
            r.start()
            rdma_x.append(r)

        sums_out = pltpu.make_async_copy(
            sums, out_hbm.at[pl.ds(base, half), :], osem.at[C])
        sums_out.start()

        other = (1 - my_x) * half
        odmas = []
        for i in range(C):
            sl = pl.ds(OFFS[i], SIZES[i])
            rdma_x[i].wait_recv()
            od = pltpu.make_async_copy(
                xrecv_buf.at[sl, :],
                out_hbm.at[pl.ds(other + OFFS[i], SIZES[i]), :],
                osem.at[i])
            od.start()
            odmas.append(od)

        sums_out.wait()
        for od in odmas:
            od.wait()
        for i in range(C):
            rdma_y[i].wait_send()
            rdma_x[i].wait_send()

    return pl.pallas_call(
        body,
        out_shape=jax.ShapeDtypeStruct((m, n), jnp.bfloat16),
        in_specs=[pl.BlockSpec(memory_space=pl.ANY)],
        out_specs=pl.BlockSpec(memory_space=pl.ANY),
        scratch_shapes=[
            pltpu.VMEM((half, n), jnp.float32),
            pltpu.VMEM((half, n), jnp.bfloat16),
            pltpu.VMEM((half, n), jnp.bfloat16),
            pltpu.VMEM((half, n), jnp.bfloat16),
            pltpu.VMEM((half, n), jnp.bfloat16),
            pltpu.SemaphoreType.DMA,
            pltpu.SemaphoreType.DMA((C + 1,)),
            pltpu.SemaphoreType.DMA((C,)),
            pltpu.SemaphoreType.DMA((C,)),
            pltpu.SemaphoreType.DMA((C,)),
            pltpu.SemaphoreType.DMA((C,)),
            pltpu.SemaphoreType.REGULAR,
        ],
        compiler_params=pltpu.CompilerParams(collective_id=0),
    )(x)


# device time: 15246 ns/iter; 1.1017x vs baseline; 1.0400x over previous
import jax
import jax.numpy as jnp
from jax import lax
from jax.experimental import pallas as pl
from jax.experimental.pallas import tpu as pltpu

SIZES = (64,) * 8
C = len(SIZES)
OFFS = tuple(sum(SIZES[:i]) for i in range(C))


def kernel(x):
    m, n = x.shape
    half = m // 2
    assert sum(SIZES) == half

    def body(x_hbm, out_hbm, xl, sendy, ybuf, xrecv_buf, sums,
             lsem, osem, ysend, yrecv, xsend, xrecv, sync_x):
        my_x = lax.axis_index("x")
        my_y = lax.axis_index("y")
        my_z = lax.axis_index("z")
        yp = (my_x, 1 - my_y, my_z)
        xn = (1 - my_x, my_y, my_z)
        base = my_x * half

        barrier_sem = pltpu.get_barrier_semaphore()
        pl.semaphore_signal(barrier_sem, inc=1, device_id=yp,
                            device_id_type=pl.DeviceIdType.MESH)
        pl.semaphore_signal(sync_x, inc=1, device_id=xn,
                            device_id_type=pl.DeviceIdType.MESH)

        stage = pltpu.make_async_copy(
            x_hbm.at[pl.ds(base, half), :], xl, lsem)
        stage.start()

        stage.wait()
        for i in range(C):
            sl = pl.ds(OFFS[i], SIZES[i])
            sendy[sl, :] = xl[sl, :].astype(jnp.bfloat16)

        pl.semaphore_wait(barrier_sem, 1)

        rdma_y = []
        for i in range(C):
            sl = pl.ds(OFFS[i], SIZES[i])
            r = pltpu.make_async_remote_copy(
                src_ref=sendy.at[sl, :], dst_ref=ybuf.at[sl, :],
                send_sem=ysend.at[i], recv_sem=yrecv.at[i],
                device_id=yp, device_id_type=pl.DeviceIdType.MESH,
            )
            r.start()
            rdma_y.append(r)

        pl.semaphore_wait(sync_x, 1)

        rdma_x = []
        for i in range(C):
            sl = pl.ds(OFFS[i], SIZES[i])
            rdma_y[i].wait_recv()
            sums[sl, :] = sendy[sl, :] + ybuf[sl, :]
            r = pltpu.make_async_remote_copy(
                src_ref=sums.at[sl, :], dst_ref=xrecv_buf.at[sl, :],
                send_sem=xsend.at[i], recv_sem=xrecv.at[i],
                device_id=xn, device_id_type=pl.DeviceIdType.MESH,
            )
            r.start()
            rdma_x.append(r)

        sums_out = pltpu.make_async_copy(
            sums, out_hbm.at[pl.ds(base, half), :], osem.at[C])
        sums_out.start()

        other = (1 - my_x) * half
        odmas = []
        for i in range(C):
            sl = pl.ds(OFFS[i], SIZES[i])
            rdma_x[i].wait_recv()
            od = pltpu.make_async_copy(
                xrecv_buf.at[sl, :],
                out_hbm.at[pl.ds(other + OFFS[i], SIZES[i]), :],
                osem.at[i])
            od.start()
            odmas.append(od)

        sums_out.wait()
        for od in odmas:
            od.wait()
        for i in range(C):
            rdma_y[i].wait_send()
            rdma_x[i].wait_send()

    return pl.pallas_call(
        body,
        out_shape=jax.ShapeDtypeStruct((m, n), jnp.bfloat16),
        in_specs=[pl.BlockSpec(memory_space=pl.ANY)],
        out_specs=pl.BlockSpec(memory_space=pl.ANY),
        scratch_shapes=[
            pltpu.VMEM((half, n), jnp.float32),
            pltpu.VMEM((half, n), jnp.bfloat16),
            pltpu.VMEM((half, n), jnp.bfloat16),
            pltpu.VMEM((half, n), jnp.bfloat16),
            pltpu.VMEM((half, n), jnp.bfloat16),
            pltpu.SemaphoreType.DMA,
            pltpu.SemaphoreType.DMA((C + 1,)),
            pltpu.SemaphoreType.DMA((C,)),
            pltpu.SemaphoreType.DMA((C,)),
            pltpu.SemaphoreType.DMA((C,)),
            pltpu.SemaphoreType.DMA((C,)),
            pltpu.SemaphoreType.REGULAR,
        ],
        compiler_params=pltpu.CompilerParams(collective_id=0),
    )(x)
